# baseline (speedup 1.0000x reference)
import functools

import jax
import jax.numpy as jnp
from jax.experimental import pallas as pl

HIDDEN = 2048
EXPERTS = 16
TOPK = 8
BLOCK = 512


def _gate_kernel(hs_ref, wt_ref, w_out_ref, i_out_ref):
    logits = jnp.dot(hs_ref[...], wt_ref[...], preferred_element_type=jnp.float32)
    lane = jax.lax.broadcasted_iota(jnp.int32, logits.shape, 1)
    vals = logits
    top_vals = []
    top_idx = []
    for _ in range(TOPK):
        m = jnp.max(vals, axis=1, keepdims=True)
        is_max = vals == m
        idx = jnp.min(jnp.where(is_max, lane, EXPERTS), axis=1, keepdims=True)
        top_vals.append(m)
        top_idx.append(idx)
        vals = jnp.where(lane == idx, -jnp.inf, vals)
    v = jnp.concatenate(top_vals, axis=1)
    e = jnp.exp(v - v[:, :1])
    w_out_ref[...] = e / jnp.sum(e, axis=1, keepdims=True)
    i_out_ref[...] = jnp.concatenate(top_idx, axis=1)


@jax.jit
def kernel(hidden_states, W):
    hs = hidden_states.reshape(-1, HIDDEN)
    n = hs.shape[0]
    wt = W.T
    grid = (n // BLOCK,)
    w_out, i_out = pl.pallas_call(
        _gate_kernel,
        grid=grid,
        in_specs=[
            pl.BlockSpec((BLOCK, HIDDEN), lambda i: (i, 0)),
            pl.BlockSpec((HIDDEN, EXPERTS), lambda i: (0, 0)),
        ],
        out_specs=[
            pl.BlockSpec((BLOCK, TOPK), lambda i: (i, 0)),
            pl.BlockSpec((BLOCK, TOPK), lambda i: (i, 0)),
        ],
        out_shape=[
            jax.ShapeDtypeStruct((n, TOPK), jnp.float32),
            jax.ShapeDtypeStruct((n, TOPK), jnp.int32),
        ],
    )(hs, wt)
    return (w_out, i_out)


# speedup vs baseline: 3.0951x; 3.0951x over previous
import jax
import jax.numpy as jnp
from jax.experimental import pallas as pl

HIDDEN = 2048
EXPERTS = 16
TOPK = 8
BLOCK = 1024


def _gate_kernel(hs_ref, w_ref, w_out_ref, i_out_ref):
    logits = jax.lax.dot_general(
        w_ref[...], hs_ref[...],
        dimension_numbers=(((1,), (1,)), ((), ())),
        preferred_element_type=jnp.float32,
    )
    sub = jax.lax.broadcasted_iota(jnp.int32, logits.shape, 0)
    vals = logits
    top_vals = []
    top_idx = []
    for _ in range(TOPK):
        m = jnp.max(vals, axis=0, keepdims=True)
        is_max = vals == m
        idx = jnp.min(jnp.where(is_max, sub, EXPERTS), axis=0, keepdims=True)
        top_vals.append(m)
        top_idx.append(idx)
        vals = jnp.where(sub == idx, -jnp.inf, vals)
    v = jnp.concatenate(top_vals, axis=0)
    e = jnp.exp(v - v[:1, :])
    w_out_ref[...] = e / jnp.sum(e, axis=0, keepdims=True)
    i_out_ref[...] = jnp.concatenate(top_idx, axis=0)


@jax.jit
def kernel(hidden_states, W):
    hs = hidden_states.reshape(-1, HIDDEN)
    n = hs.shape[0]
    grid = (n // BLOCK,)
    w_out, i_out = pl.pallas_call(
        _gate_kernel,
        grid=grid,
        in_specs=[
            pl.BlockSpec((BLOCK, HIDDEN), lambda i: (i, 0)),
            pl.BlockSpec((EXPERTS, HIDDEN), lambda i: (0, 0)),
        ],
        out_specs=[
            pl.BlockSpec((TOPK, BLOCK), lambda i: (0, i)),
            pl.BlockSpec((TOPK, BLOCK), lambda i: (0, i)),
        ],
        out_shape=[
            jax.ShapeDtypeStruct((TOPK, n), jnp.float32),
            jax.ShapeDtypeStruct((TOPK, n), jnp.int32),
        ],
    )(hs, W)
    return (w_out.T, i_out.T)
